# baseline (speedup 1.0000x reference)
import functools

import jax
import jax.numpy as jnp
import numpy as np
from jax import lax
from jax.experimental import pallas as pl
from jax.experimental.pallas import tpu as pltpu
from jax.experimental.pallas import tpu_sc as plsc

N = 10000
D = 128
E = 320000
NSEED = 3000
NG = 2
NSC = 2
NSUB = 16
NP = 10240
CH = 128
EE = 2 * E
NCH = 316
EPT = NCH * CH
EP = NSUB * EPT
RPT = NP // NSUB
BLK = 256
GB = NP // BLK
SPAD = 3072
SB = 96

_mesh = plsc.VectorSubcoreMesh(core_axis_name="c", subcore_axis_name="s",
                               num_cores=NSC, num_subcores=NSUB)

DW = D // 2


@functools.partial(
    pl.kernel, mesh=_mesh,
    out_type=jax.ShapeDtypeStruct((NG, NSUB, NP), jnp.float32),
    compiler_params=pltpu.CompilerParams(needs_layout_passes=False),
    scratch_types=[
        pltpu.VMEM((EPT,), jnp.int32),
        pltpu.VMEM((NP,), jnp.float32),
    ])
def _deg_call(dst_hbm, zeros_hbm, out_hbm, idx_v, hist_v):
    c = lax.axis_index("c")
    s = lax.axis_index("s")
    pltpu.sync_copy(zeros_hbm, hist_v)
    base = c * EP + s * EPT
    pltpu.sync_copy(dst_hbm.at[pl.ds(base, EPT)], idx_v)
    ones = jnp.full((16,), 1.0, jnp.float32)

    def chunk(i, carry):
        off = i * CH
        for j in range(CH // 16):
            idx = idx_v[pl.ds(off + j * 16, 16)]
            plsc.addupdate_scatter(hist_v, [idx], ones)
        return carry

    lax.fori_loop(0, NCH, chunk, 0)
    pltpu.sync_copy(hist_v, out_hbm.at[c, s])


@functools.partial(
    pl.kernel, mesh=_mesh,
    out_type=jax.ShapeDtypeStruct((NG * NP, D), jnp.float32),
    compiler_params=pltpu.CompilerParams(needs_layout_passes=False,
                                         use_tc_tiling_on_sc=False),
    scratch_types=[
        pltpu.VMEM_SHARED((NP, D), jnp.float32),
        [pltpu.VMEM((2, CH), jnp.int32)] * 4,
        [pltpu.VMEM((CH, DW), jnp.int32)] * 2,
        pltpu.VMEM((CH, D), jnp.float32),
        [pltpu.SemaphoreType.DMA] * 4,
        [pltpu.SemaphoreType.DMA] * 2,
        pltpu.SemaphoreType.DMA,
    ])
def _mp_call(idx2_hbm, tab_hbm, zrow_hbm, out_hbm,
             acc, ib, rbf, rf, isem, gsem, ssem):
    c = lax.axis_index("c")
    s = lax.axis_index("s")
    r0 = s * RPT
    pltpu.sync_copy(zrow_hbm, rf)
    for j in range(RPT // CH):
        pltpu.sync_copy(rf, acc.at[pl.ds(r0 + j * CH, CH)])
    plsc.subcore_barrier()
    g0 = (c * NSUB + s) * NCH

    def wait_gather(sem):
        pltpu.make_async_copy(tab_hbm.at[pl.ds(0, CH)], rbf[0], sem).wait()

    def wait_scatter():
        pltpu.make_async_copy(zrow_hbm, rf, ssem).wait()

    def wait_idx(sem):
        pltpu.make_async_copy(idx2_hbm.at[g0], ib[0], sem).wait()

    for t in range(3):
        pltpu.async_copy(idx2_hbm.at[g0 + t], ib[t], isem[t])
    wait_idx(isem[0])
    pltpu.async_copy(tab_hbm.at[ib[0].at[0]], rbf[0], gsem[0])

    def body(b, carry):
        for t in range(4):
            i = 4 * b + t
            p = t % 2

            @pl.when(i + 1 < NCH)
            def _():
                wait_idx(isem[(t + 1) % 4])
                pltpu.async_copy(tab_hbm.at[ib[(t + 1) % 4].at[0]],
                                 rbf[1 - p], gsem[1 - p])

            @pl.when(i + 3 < NCH)
            def _():
                pltpu.async_copy(idx2_hbm.at[g0 + i + 3], ib[(t + 3) % 4],
                                 isem[(t + 3) % 4])
            wait_gather(gsem[p])

            @pl.when(i > 0)
            def _():
                wait_scatter()

            def conv(r4, cc):
                for rr in range(4):
                    r = 4 * r4 + rr
                    for jj in range(DW // 16):
                        w = rbf[p][r, pl.ds(jj * 16, 16)]
                        xb = plsc.bitcast(w, jnp.bfloat16)
                        a, bb = plsc.unpack(
                            xb, format=plsc.PackFormat.INTERLEAVED)
                        rf[r, pl.ds(jj * 16, 16)] = a
                        rf[r, pl.ds(DW + jj * 16, 16)] = bb
                return cc

            lax.fori_loop(0, CH // 4, conv, 0)
            pltpu.async_copy(rf, acc.at[ib[t].at[1]], ssem, add=True)
        return carry

    lax.fori_loop(0, NCH // 4, body, 0)
    wait_scatter()
    plsc.subcore_barrier()
    for j in range(RPT // CH):
        pltpu.sync_copy(acc.at[pl.ds(r0 + j * CH, CH)], rf)
        pltpu.sync_copy(rf, out_hbm.at[pl.ds(c * NP + r0 + j * CH, CH)])


@functools.partial(
    pl.kernel, mesh=_mesh,
    out_type=jax.ShapeDtypeStruct((NG * SPAD, D), jnp.float32),
    scratch_types=[
        pltpu.VMEM((SB,), jnp.int32),
        pltpu.VMEM((SB, D), jnp.float32),
        pltpu.SemaphoreType.DMA,
    ])
def _seed_call(seed_hbm, tab_hbm, out_hbm, idx_v, rows_v, sem):
    c = lax.axis_index("c")
    s = lax.axis_index("s")
    w = s * NSC + c
    base = w * 2 * SB
    for j in range(2):
        pltpu.sync_copy(seed_hbm.at[pl.ds(base + j * SB, SB)], idx_v)
        pltpu.async_copy(tab_hbm.at[idx_v], rows_v, sem).wait()
        pltpu.sync_copy(rows_v, out_hbm.at[pl.ds(base + j * SB, SB)])


def _dinv(dg_ref):
    cnt = jnp.sum(dg_ref[0], axis=0)
    return lax.rsqrt(cnt + 1.0)


def _dot(a, b):
    return jnp.dot(a, b, preferred_element_type=jnp.float32,
                   precision=lax.Precision.HIGHEST)


def _pack_i32(x):
    xb = x.astype(jnp.bfloat16)
    lo = lax.bitcast_convert_type(xb[:, :DW], jnp.uint16).astype(jnp.uint32)
    hi = lax.bitcast_convert_type(xb[:, DW:], jnp.uint16).astype(jnp.uint32)
    return lax.bitcast_convert_type(lo | (hi << 16), jnp.int32)


def _mm1_body(h_ref, wt_ref, dg_ref, o_ref):
    dinv = _dinv(dg_ref)
    o_ref[...] = _pack_i32(_dot(h_ref[...], wt_ref[...]) * dinv[:, None])


def _fin1_body(p_ref, h0_ref, dg_ref, w0_ref, w1t_ref, h1_ref, o2_ref):
    dinv = _dinv(dg_ref)
    hws1 = _dot(h0_ref[...], w0_ref[...]) * dinv[:, None]
    agg = (p_ref[...] + hws1) * dinv[:, None]
    h1 = jnp.maximum(agg, 0.0) + h0_ref[...]
    h1_ref[...] = h1
    o2_ref[...] = _pack_i32(_dot(h1, w1t_ref[...]) * dinv[:, None])


def _fin2_body(p_ref, h1_ref, dg_ref, w1_ref, o_ref):
    dinv = _dinv(dg_ref)
    hws2 = _dot(h1_ref[...], w1_ref[...]) * dinv[:, None]
    agg = (p_ref[...] + hws2) * dinv[:, None]
    o_ref[...] = jnp.maximum(agg, 0.0) + h1_ref[...]


_row_spec = pl.BlockSpec((BLK, D), lambda i: (i, 0))
_w_spec = pl.BlockSpec((D, D), lambda i: (0, 0))
_deg_spec = pl.BlockSpec((1, NSUB, BLK), lambda i: (i // GB, 0, i % GB))
_pk_spec = pl.BlockSpec((BLK, DW), lambda i: (i, 0))
_rows_out = jax.ShapeDtypeStruct((NG * NP, D), jnp.float32)
_rows_out_pk = jax.ShapeDtypeStruct((NG * NP, DW), jnp.int32)


def _mm1(h0, w0, deg):
    return pl.pallas_call(
        _mm1_body, grid=(NG * GB,),
        in_specs=[_row_spec, _w_spec, _deg_spec],
        out_specs=_pk_spec, out_shape=_rows_out_pk)(h0, w0, deg)


def _fin1(p1, h0, deg, w0, w1):
    return pl.pallas_call(
        _fin1_body, grid=(NG * GB,),
        in_specs=[_row_spec, _row_spec, _deg_spec, _w_spec, _w_spec],
        out_specs=(_row_spec, _pk_spec),
        out_shape=(_rows_out, _rows_out_pk))(p1, h0, deg, w0, w1)


def _fin2(p2, h1, deg, w1):
    return pl.pallas_call(
        _fin2_body, grid=(NG * GB,),
        in_specs=[_row_spec, _row_spec, _deg_spec, _w_spec],
        out_specs=_row_spec, out_shape=_rows_out)(p2, h1, deg, w1)


def kernel(sr_ent_seeds, tg_ent_seeds, triples_sr, triples_tg,
           embedding_sr, embedding_tg, edges_sr, edges_tg, W0, W1):
    f32 = jnp.float32
    i32 = jnp.int32

    def prep_edges(edges, goff):
        s0 = edges[:, 0]
        d0 = edges[:, 1]
        pad = N + (jnp.arange(EP - EE, dtype=i32) % (NP - N))
        src = jnp.concatenate([s0, d0, pad]) + goff
        dst = jnp.concatenate([d0, s0, pad])
        return src, dst

    src_a, dst_a = prep_edges(edges_sr, 0)
    src_b, dst_b = prep_edges(edges_tg, NP)
    src_all = jnp.concatenate([src_a, src_b])
    dst_all = jnp.concatenate([dst_a, dst_b])
    idx2 = jnp.stack([src_all.reshape(NG * NSUB * NCH, CH),
                      dst_all.reshape(NG * NSUB * NCH, CH)], axis=1)

    zpad = jnp.zeros((NP - N, D), f32)
    h0 = jnp.concatenate([embedding_sr, zpad, embedding_tg, zpad])
    zhist = jnp.zeros((NP,), f32)
    zrow = jnp.zeros((CH, D), f32)

    deg = _deg_call(dst_all, zhist)
    tab1 = _mm1(h0, W0, deg)
    p1 = _mp_call(idx2, tab1, zrow)
    h1, tab2 = _fin1(p1, h0, deg, W0, W1)
    p2 = _mp_call(idx2, tab2, zrow)
    h2 = _fin2(p2, h1, deg, W1)

    spad = jnp.zeros((SPAD - NSEED,), i32)
    seeds = jnp.concatenate([sr_ent_seeds, spad, tg_ent_seeds + NP, spad])
    sg = _seed_call(seeds, h2)

    return (sg[0:NSEED], sg[SPAD:SPAD + NSEED], h2[0:N], h2[NP:NP + N])


# speedup vs baseline: 2.1415x; 2.1415x over previous
import functools

import jax
import jax.numpy as jnp
from jax import lax
from jax.experimental import pallas as pl
from jax.experimental.pallas import tpu as pltpu
from jax.experimental.pallas import tpu_sc as plsc

N = 10000
D = 128
E = 320000
NSEED = 3000
NG = 2
NSC = 2
NSUB = 16
NP = 10240
CH = 128
EE = 2 * E
NCH = 316
EPT = NCH * CH
EP = NSUB * EPT
RPT = NP // NSUB
BLK = 256
GB = NP // BLK
SPAD = 3072
SB = 96

_mesh = plsc.VectorSubcoreMesh(core_axis_name="c", subcore_axis_name="s",
                               num_cores=NSC, num_subcores=NSUB)


@functools.partial(
    pl.kernel, mesh=_mesh,
    out_type=jax.ShapeDtypeStruct((NG, NSUB, NP), jnp.float32),
    compiler_params=pltpu.CompilerParams(needs_layout_passes=False),
    scratch_types=[
        pltpu.VMEM((EPT,), jnp.int32),
        pltpu.VMEM((NP,), jnp.float32),
    ])
def _deg_call(dst_hbm, zeros_hbm, out_hbm, idx_v, hist_v):
    c = lax.axis_index("c")
    s = lax.axis_index("s")
    pltpu.sync_copy(zeros_hbm, hist_v)
    base = c * EP + s * EPT
    pltpu.sync_copy(dst_hbm.at[pl.ds(base, EPT)], idx_v)
    ones = jnp.full((16,), 1.0, jnp.float32)

    def chunk(i, carry):
        off = i * CH
        for j in range(CH // 16):
            idx = idx_v[pl.ds(off + j * 16, 16)]
            plsc.addupdate_scatter(hist_v, [idx], ones)
        return carry

    lax.fori_loop(0, NCH, chunk, 0)
    pltpu.sync_copy(hist_v, out_hbm.at[c, s])


@functools.partial(
    pl.kernel, mesh=_mesh,
    out_type=jax.ShapeDtypeStruct((NG * NP, D), jnp.float32),
    scratch_types=[
        pltpu.VMEM_SHARED((NP, D), jnp.float32),
        [pltpu.VMEM((2, CH), jnp.int32)] * 4,
        [pltpu.VMEM((CH, D), jnp.float32)] * 2,
        [pltpu.SemaphoreType.DMA] * 4,
        [pltpu.SemaphoreType.DMA] * 2,
        [pltpu.SemaphoreType.DMA] * 2,
    ])
def _mp_call(idx2_hbm, tab_hbm, zrow_hbm, out_hbm,
             acc, ib, rows, isem, gsem, ssem):
    c = lax.axis_index("c")
    s = lax.axis_index("s")
    r0 = s * RPT
    pltpu.sync_copy(zrow_hbm, rows[0])
    for j in range(RPT // CH):
        pltpu.sync_copy(rows[0], acc.at[pl.ds(r0 + j * CH, CH)])
    plsc.subcore_barrier()
    g0 = (c * NSUB + s) * NCH

    def wait_rows(sem):
        pltpu.make_async_copy(zrow_hbm, rows[0], sem).wait()

    def wait_idx(sem):
        pltpu.make_async_copy(idx2_hbm.at[g0], ib[0], sem).wait()

    for t in range(3):
        pltpu.async_copy(idx2_hbm.at[g0 + t], ib[t], isem[t])
    wait_idx(isem[0])
    pltpu.async_copy(tab_hbm.at[ib[0].at[0]], rows[0], gsem[0])

    def body(b, carry):
        for t in range(4):
            i = 4 * b + t
            p = t % 2

            @pl.when(i > 0)
            def _():
                wait_rows(ssem[1 - p])

            @pl.when(i + 1 < NCH)
            def _():
                wait_idx(isem[(t + 1) % 4])
                pltpu.async_copy(tab_hbm.at[ib[(t + 1) % 4].at[0]],
                                 rows[1 - p], gsem[1 - p])

            @pl.when(i + 3 < NCH)
            def _():
                pltpu.async_copy(idx2_hbm.at[g0 + i + 3], ib[(t + 3) % 4],
                                 isem[(t + 3) % 4])
            wait_rows(gsem[p])
            pltpu.async_copy(rows[p], acc.at[ib[t].at[1]], ssem[p], add=True)
        return carry

    lax.fori_loop(0, NCH // 4, body, 0)
    wait_rows(ssem[1])
    plsc.subcore_barrier()
    for j in range(RPT // CH):
        pltpu.sync_copy(acc.at[pl.ds(r0 + j * CH, CH)], rows[0])
        pltpu.sync_copy(rows[0], out_hbm.at[pl.ds(c * NP + r0 + j * CH, CH)])


@functools.partial(
    pl.kernel, mesh=_mesh,
    out_type=jax.ShapeDtypeStruct((NG * SPAD, D), jnp.float32),
    scratch_types=[
        pltpu.VMEM((SB,), jnp.int32),
        pltpu.VMEM((SB, D), jnp.float32),
        pltpu.SemaphoreType.DMA,
    ])
def _seed_call(seed_hbm, tab_hbm, out_hbm, idx_v, rows_v, sem):
    c = lax.axis_index("c")
    s = lax.axis_index("s")
    w = s * NSC + c
    base = w * 2 * SB
    for j in range(2):
        pltpu.sync_copy(seed_hbm.at[pl.ds(base + j * SB, SB)], idx_v)
        pltpu.async_copy(tab_hbm.at[idx_v], rows_v, sem).wait()
        pltpu.sync_copy(rows_v, out_hbm.at[pl.ds(base + j * SB, SB)])


def _dinv(dg_ref):
    cnt = jnp.sum(dg_ref[0], axis=0)
    return lax.rsqrt(cnt + 1.0)


def _mm1_body(h_ref, w_ref, dg_ref, o_ref):
    dinv = _dinv(dg_ref)
    hw = jnp.dot(h_ref[...], w_ref[...], preferred_element_type=jnp.float32,
                 precision=lax.Precision.HIGHEST)
    o_ref[...] = hw * dinv[:, None]


def _fin1_body(p_ref, hws_ref, h0_ref, dg_ref, w_ref, h1_ref, o2_ref):
    dinv = _dinv(dg_ref)
    agg = (p_ref[...] + hws_ref[...]) * dinv[:, None]
    h1 = jnp.maximum(agg, 0.0) + h0_ref[...]
    h1_ref[...] = h1
    o2_ref[...] = jnp.dot(h1, w_ref[...], preferred_element_type=jnp.float32,
                          precision=lax.Precision.HIGHEST) * dinv[:, None]


def _fin2_body(p_ref, hws_ref, h1_ref, dg_ref, o_ref):
    dinv = _dinv(dg_ref)
    agg = (p_ref[...] + hws_ref[...]) * dinv[:, None]
    o_ref[...] = jnp.maximum(agg, 0.0) + h1_ref[...]


_row_spec = pl.BlockSpec((BLK, D), lambda i: (i, 0))
_w_spec = pl.BlockSpec((D, D), lambda i: (0, 0))
_deg_spec = pl.BlockSpec((1, NSUB, BLK), lambda i: (i // GB, 0, i % GB))
_rows_out = jax.ShapeDtypeStruct((NG * NP, D), jnp.float32)


def _mm1(h0, w0, deg):
    return pl.pallas_call(
        _mm1_body, grid=(NG * GB,),
        in_specs=[_row_spec, _w_spec, _deg_spec],
        out_specs=_row_spec, out_shape=_rows_out)(h0, w0, deg)


def _fin1(p1, hws1, h0, deg, w1):
    return pl.pallas_call(
        _fin1_body, grid=(NG * GB,),
        in_specs=[_row_spec, _row_spec, _row_spec, _deg_spec, _w_spec],
        out_specs=(_row_spec, _row_spec),
        out_shape=(_rows_out, _rows_out))(p1, hws1, h0, deg, w1)


def _fin2(p2, hws2, h1, deg):
    return pl.pallas_call(
        _fin2_body, grid=(NG * GB,),
        in_specs=[_row_spec, _row_spec, _row_spec, _deg_spec],
        out_specs=_row_spec, out_shape=_rows_out)(p2, hws2, h1, deg)


def kernel(sr_ent_seeds, tg_ent_seeds, triples_sr, triples_tg,
           embedding_sr, embedding_tg, edges_sr, edges_tg, W0, W1):
    f32 = jnp.float32
    i32 = jnp.int32

    def prep_edges(edges, goff):
        s0 = edges[:, 0]
        d0 = edges[:, 1]
        pad = N + (jnp.arange(EP - EE, dtype=i32) % (NP - N))
        src = jnp.concatenate([s0, d0, pad]) + goff
        dst = jnp.concatenate([d0, s0, pad])
        return src, dst

    src_a, dst_a = prep_edges(edges_sr, 0)
    src_b, dst_b = prep_edges(edges_tg, NP)
    src_all = jnp.concatenate([src_a, src_b])
    dst_all = jnp.concatenate([dst_a, dst_b])
    idx2 = jnp.stack([src_all.reshape(NG * NSUB * NCH, CH),
                      dst_all.reshape(NG * NSUB * NCH, CH)], axis=1)

    zpad = jnp.zeros((NP - N, D), f32)
    h0 = jnp.concatenate([embedding_sr, zpad, embedding_tg, zpad])
    zhist = jnp.zeros((NP,), f32)
    zrow = jnp.zeros((CH, D), f32)

    deg = _deg_call(dst_all, zhist)
    hws1 = _mm1(h0, W0, deg)
    p1 = _mp_call(idx2, hws1, zrow)
    h1, hws2 = _fin1(p1, hws1, h0, deg, W1)
    p2 = _mp_call(idx2, hws2, zrow)
    h2 = _fin2(p2, hws2, h1, deg)

    spad = jnp.zeros((SPAD - NSEED,), i32)
    seeds = jnp.concatenate([sr_ent_seeds, spad, tg_ent_seeds + NP, spad])
    sg = _seed_call(seeds, h2)

    return (sg[0:NSEED], sg[SPAD:SPAD + NSEED], h2[0:N], h2[NP:NP + N])
